# baseline (speedup 1.0000x reference)
import functools

import jax
import jax.numpy as jnp
from jax import lax
from jax.experimental import pallas as pl
from jax.experimental.pallas import tpu as pltpu
from jax.experimental.pallas import tpu_sc as plsc

D_MODEL = 1024
MAX_LEN = 8192
BATCH = 4
SEQ = 4096

NUM_CORES = 2
NUM_SUBCORES = 16
LANES = 16
NW = NUM_CORES * NUM_SUBCORES

ROWS_PER_W = SEQ // NW
CHUNK = 16
NCHUNK = ROWS_PER_W // CHUNK
VECS = D_MODEL // LANES
NBX = 5
LOOKAHEAD = 2


def _build_sc_kernel():
    mesh = plsc.VectorSubcoreMesh(
        core_axis_name="c", subcore_axis_name="s", num_cores=NUM_CORES
    )

    @functools.partial(
        pl.kernel,
        mesh=mesh,
        out_type=jax.ShapeDtypeStruct((BATCH, SEQ, D_MODEL), jnp.float32),
        scratch_types=[
            pltpu.VMEM((ROWS_PER_W,), jnp.int32),
            [pltpu.VMEM((CHUNK, D_MODEL), jnp.float32)] * 2,
            [pltpu.VMEM((CHUNK, D_MODEL), jnp.float32)] * NBX,
            [pltpu.SemaphoreType.DMA] * 2,
            [pltpu.SemaphoreType.DMA] * NBX,
            [pltpu.SemaphoreType.DMA] * NBX,
        ],
    )
    def sc_kernel(x_hbm, idx_hbm, pe_hbm, out_hbm,
                  idx_v, pe_bufs, x_bufs, gsems, xsems, osems):
        wid = lax.axis_index("s") * NUM_CORES + lax.axis_index("c")
        base = wid * ROWS_PER_W
        pltpu.sync_copy(idx_hbm.at[pl.ds(base, ROWS_PER_W)], idx_v)

        steps = [(c, b) for c in range(NCHUNK) for b in range(BATCH)]
        T = len(steps)

        def start_gather(c):
            return pltpu.async_copy(
                pe_hbm.at[idx_v.at[pl.ds(c * CHUNK, CHUNK)]],
                pe_bufs[c % 2], gsems[c % 2],
            )

        def start_xload(t):
            c, b = steps[t]
            return pltpu.async_copy(
                x_hbm.at[b, pl.ds(base + c * CHUNK, CHUNK)],
                x_bufs[t % NBX], xsems[t % NBX],
            )

        gather_d = {0: start_gather(0)}
        xload_d = {t: start_xload(t) for t in range(LOOKAHEAD)}
        store_d = {}

        for t, (c, b) in enumerate(steps):
            xbuf = t % NBX
            pe_v = pe_bufs[c % 2]
            x_v = x_bufs[xbuf]

            if b == 0:
                if c + 1 < NCHUNK:
                    gather_d[c + 1] = start_gather(c + 1)
                gather_d[c].wait()

            ta = t + LOOKAHEAD
            if ta < T:
                if ta >= NBX:
                    for d in store_d[ta - NBX]:
                        d.wait()
                xload_d[ta] = start_xload(ta)

            xload_d[t].wait()

            half = CHUNK // 2
            halves = []
            for h in range(2):
                r0 = h * half

                @plsc.parallel_loop(0, half * VECS, unroll=8)
                def _add(i, r0=r0):
                    r = r0 + (i >> 6)
                    col = pl.multiple_of((i & (VECS - 1)) << 4, LANES)
                    plsc.addupdate(
                        x_v.at[r, pl.ds(col, LANES)],
                        pe_v[r, pl.ds(col, LANES)],
                    )

                halves.append(pltpu.async_copy(
                    x_v.at[pl.ds(r0, half)],
                    out_hbm.at[b, pl.ds(base + c * CHUNK + r0, half)],
                    osems[xbuf],
                ))
            store_d[t] = halves

        for t in range(T - NBX, T):
            for d in store_d[t]:
                d.wait()

    return sc_kernel


_sc_kernel = _build_sc_kernel()


@jax.jit
def kernel(x, indices, pe):
    pe2d = pe.reshape(MAX_LEN, D_MODEL)
    return _sc_kernel(x, indices, pe2d)


# speedup vs baseline: 1.0395x; 1.0395x over previous
import functools

import jax
import jax.numpy as jnp
from jax import lax
from jax.experimental import pallas as pl
from jax.experimental.pallas import tpu as pltpu
from jax.experimental.pallas import tpu_sc as plsc

D_MODEL = 1024
MAX_LEN = 8192
BATCH = 4
SEQ = 4096

NUM_CORES = 2
NUM_SUBCORES = 16
LANES = 16
NW = NUM_CORES * NUM_SUBCORES

ROWS_PER_W = SEQ // NW
CHUNK = 16
NCHUNK = ROWS_PER_W // CHUNK
VECS = D_MODEL // LANES
NBX = 5
LOOKAHEAD = 2


def _build_sc_kernel():
    mesh = plsc.VectorSubcoreMesh(
        core_axis_name="c", subcore_axis_name="s", num_cores=NUM_CORES
    )

    @functools.partial(
        pl.kernel,
        mesh=mesh,
        out_type=jax.ShapeDtypeStruct((BATCH, SEQ, D_MODEL), jnp.float32),
        scratch_types=[
            pltpu.VMEM((ROWS_PER_W,), jnp.int32),
            [pltpu.VMEM((CHUNK, D_MODEL), jnp.float32)] * 2,
            [pltpu.VMEM((CHUNK, D_MODEL), jnp.float32)] * NBX,
            [pltpu.SemaphoreType.DMA] * 2,
            [pltpu.SemaphoreType.DMA] * NBX,
            [pltpu.SemaphoreType.DMA] * NBX,
        ],
    )
    def sc_kernel(x_hbm, idx_hbm, pe_hbm, out_hbm,
                  idx_v, pe_bufs, x_bufs, gsems, xsems, osems):
        wid = lax.axis_index("s") * NUM_CORES + lax.axis_index("c")
        base = wid * ROWS_PER_W
        pltpu.sync_copy(idx_hbm.at[pl.ds(base, ROWS_PER_W)], idx_v)

        steps = [(c, b) for c in range(NCHUNK) for b in range(BATCH)]
        T = len(steps)

        def start_gather(c):
            return pltpu.async_copy(
                pe_hbm.at[idx_v.at[pl.ds(c * CHUNK, CHUNK)]],
                pe_bufs[c % 2], gsems[c % 2],
            )

        def start_xload(t):
            c, b = steps[t]
            return pltpu.async_copy(
                x_hbm.at[b, pl.ds(base + c * CHUNK, CHUNK)],
                x_bufs[t % NBX], xsems[t % NBX],
            )

        gather_d = {0: start_gather(0)}
        xload_d = {t: start_xload(t) for t in range(LOOKAHEAD)}
        store_d = {}

        for t, (c, b) in enumerate(steps):
            xbuf = t % NBX
            pe_v = pe_bufs[c % 2]
            x_v = x_bufs[xbuf]

            if b == 0:
                if c + 1 < NCHUNK:
                    gather_d[c + 1] = start_gather(c + 1)
                gather_d[c].wait()

            ta = t + LOOKAHEAD
            if ta < T:
                if ta >= NBX:
                    store_d[ta - NBX].wait()
                xload_d[ta] = start_xload(ta)

            xload_d[t].wait()

            @plsc.parallel_loop(0, CHUNK * VECS, unroll=8)
            def _add(i):
                r = i >> 6
                col = pl.multiple_of((i & (VECS - 1)) << 4, LANES)
                plsc.addupdate(
                    x_v.at[r, pl.ds(col, LANES)], pe_v[r, pl.ds(col, LANES)]
                )

            store_d[t] = pltpu.async_copy(
                x_v, out_hbm.at[b, pl.ds(base + c * CHUNK, CHUNK)],
                osems[xbuf],
            )

        for t in range(T - NBX, T):
            store_d[t].wait()

    return sc_kernel


_sc_kernel = _build_sc_kernel()


@jax.jit
def kernel(x, indices, pe):
    pe2d = pe.reshape(MAX_LEN, D_MODEL)
    return _sc_kernel(x, indices, pe2d)
